# baseline (speedup 1.0000x reference)
import functools
import math

import jax
import jax.numpy as jnp
from jax import lax
from jax.experimental import pallas as pl
from jax.experimental.pallas import tpu as pltpu
from jax.experimental.pallas import tpu_sc as plsc

_D = 64
_S2_BITS = 20
_S2_MASK = (1 << _S2_BITS) - 1
_NW = 32
_L = 200
_BPW = 128


def _proj_t_kernel(embt_ref, w_ref, out_ref):
    out_ref[...] = lax.dot_general(embt_ref[...], w_ref[...],
                                   (((0,), (0,)), ((), ())),
                                   preferred_element_type=jnp.float32)


def _proj_t_bias_kernel(embt_ref, w_ref, b_ref, out_ref):
    out_ref[...] = lax.dot_general(embt_ref[...], w_ref[...],
                                   (((0,), (0,)), ((), ())),
                                   preferred_element_type=jnp.float32) + b_ref[...]


@functools.lru_cache(maxsize=None)
def _make_sc_gather(batch: int, seq: int):
    assert batch == _NW * _BPW and seq == _L
    tok_per_w = _BPW * _L
    mesh = plsc.VectorSubcoreMesh(core_axis_name="c", subcore_axis_name="s")

    _ST = 65

    @functools.partial(
        pl.kernel,
        out_type=jax.ShapeDtypeStruct((_L, _D // 8, _NW, 8 * _BPW), jnp.float32),
        mesh=mesh,
        compiler_params=pltpu.CompilerParams(use_tc_tiling_on_sc=False,
                                             needs_layout_passes=False),
        scratch_types=[
            pltpu.VMEM((tok_per_w,), jnp.int32),
            pltpu.VMEM((_BPW,), jnp.int32),
            pltpu.VMEM((2, _BPW), jnp.int32),
            pltpu.VMEM((2, _BPW), jnp.int32),
            pltpu.VMEM((2, _BPW, _D), jnp.float32),
            pltpu.VMEM((2, _BPW, _D), jnp.float32),
            pltpu.VMEM((_BPW * _ST,), jnp.float32),
            pltpu.VMEM((2, _D * _BPW), jnp.float32),
            pltpu.SemaphoreType.DMA,
            pltpu.SemaphoreType.DMA,
            pltpu.SemaphoreType.DMA,
            pltpu.SemaphoreType.DMA,
        ],
    )
    def sc_gather(tid_hbm, p1_hbm, p2_hbm, out_hbm,
                  tid_v, bl_v, s1b, s2b, g1b, g2b, sump, stg,
                  sg0, sg1, sw0, sw1):
        wid = lax.axis_index("s") * 2 + lax.axis_index("c")
        sems_g = (sg0, sg1)
        sems_w = (sw0, sw1)

        pltpu.sync_copy(tid_hbm.at[pl.ds(wid * tok_per_w, tok_per_w)], tid_v)
        for k in range(_BPW // 16):
            bl_v[pl.ds(k * 16, 16)] = (lax.iota(jnp.int32, 16) + k * 16) * _L

        def extract(l, par):
            for k in range(_BPW // 16):
                sl = pl.ds(k * 16, 16)
                t = plsc.load_gather(tid_v, [bl_v[sl] + l])
                s1b[par, sl] = lax.shift_right_logical(t, _S2_BITS)
                s2b[par, sl] = (t & _S2_MASK) * 2

        def start_gather(par):
            pltpu.async_copy(p1_hbm.at[s1b.at[par]], g1b.at[par], sems_g[par])
            pltpu.async_copy(p2_hbm.at[s2b.at[par]], g2b.at[par], sems_g[par])

        def wait_gather(par):
            pltpu.make_async_copy(p1_hbm.at[s1b.at[par]], g1b.at[par],
                                  sems_g[par]).wait()
            pltpu.make_async_copy(p2_hbm.at[s2b.at[par]], g2b.at[par],
                                  sems_g[par]).wait()

        def repack(par):
            @plsc.parallel_loop(0, _BPW, unroll=8)
            def _(r):
                for k in range(_D // 16):
                    sl = pl.ds(k * 16, 16)
                    sump[pl.ds(r * _ST + k * 16, 16)] = (
                        g1b[par, r, sl] + g2b[par, r, sl])

        def issue_writes(l, par):
            for jg in range(_D // 8):
                pltpu.async_copy(
                    stg.at[par].at[pl.ds(jg * 8 * _BPW, 8 * _BPW)],
                    out_hbm.at[l, jg, wid], sems_w[par])

        def wait_writes(l, par):
            for jg in range(_D // 8):
                pltpu.make_async_copy(
                    stg.at[par].at[pl.ds(jg * 8 * _BPW, 8 * _BPW)],
                    out_hbm.at[l, jg, wid], sems_w[par]).wait()

        def assemble(l, par):
            iota = lax.iota(jnp.int32, 16)

            @plsc.parallel_loop(0, _D, unroll=8)
            def _(j):
                jsp = jnp.full((16,), j, jnp.int32)
                for bs in range(_BPW // 16):
                    idxv = (iota + bs * 16) * _ST + jsp
                    stg[par, pl.ds(j * _BPW + bs * 16, 16)] = (
                        plsc.load_gather(sump, [idxv]))

        extract(0, 0)
        start_gather(0)

        def pair_body(lp, carry):
            l0 = lp * 2
            l1 = l0 + 1
            wait_gather(0)
            extract(l1, 1)
            start_gather(1)
            repack(0)

            @pl.when(lp > 0)
            def _():
                wait_writes(l0, 0)

            assemble(l0, 0)
            issue_writes(l0, 0)

            wait_gather(1)

            @pl.when(lp < _L // 2 - 1)
            def _():
                extract(l1 + 1, 0)
                start_gather(0)

            repack(1)

            @pl.when(lp > 0)
            def _():
                wait_writes(l1, 1)

            assemble(l1, 1)
            issue_writes(l1, 1)
            return carry

        lax.fori_loop(0, _L // 2, pair_body, 0)
        wait_writes(_L - 2, 0)
        wait_writes(_L - 1, 1)

    return sc_gather


def kernel(token_ids, emb_s1, emb_s2, W, b):
    B, L = token_ids.shape
    n = B * L
    n1 = emb_s1.shape[0]
    n2 = emb_s2.shape[0]
    scale = math.sqrt(_D)
    w1 = W[:_D] * scale
    w2 = jnp.concatenate([W[_D:] * scale,
                          jnp.zeros((_D, _D), jnp.float32)], axis=1)

    p1 = pl.pallas_call(
        _proj_t_bias_kernel,
        out_shape=jax.ShapeDtypeStruct((n1, _D), jnp.float32),
    )(emb_s1.T, w1, b.reshape(1, _D))

    blk = 8192
    p2 = pl.pallas_call(
        _proj_t_kernel,
        grid=(n2 // blk,),
        in_specs=[
            pl.BlockSpec((_D, blk), lambda i: (0, i)),
            pl.BlockSpec((_D, 2 * _D), lambda i: (0, 0)),
        ],
        out_specs=pl.BlockSpec((blk, 2 * _D), lambda i: (i, 0)),
        out_shape=jax.ShapeDtypeStruct((n2, 2 * _D), jnp.float32),
    )(emb_s2.T, w2)

    o = _make_sc_gather(B, L)(
        token_ids.reshape(n),
        p1,
        p2.reshape(n2 * 2, _D),
    )
    o5 = o.reshape(_L, _D // 8, _NW, 8, _BPW)
    return o5.transpose(2, 4, 0, 1, 3).reshape(B, L, _D)


# speedup vs baseline: 1.0386x; 1.0386x over previous
import functools
import math

import jax
import jax.numpy as jnp
from jax import lax
from jax.experimental import pallas as pl
from jax.experimental.pallas import tpu as pltpu
from jax.experimental.pallas import tpu_sc as plsc

_D = 64
_S2_BITS = 20
_S2_MASK = (1 << _S2_BITS) - 1
_NW = 32
_L = 200
_BPW = 128


def _proj_t_kernel(embt_ref, w_ref, out_ref):
    out_ref[...] = lax.dot_general(embt_ref[...], w_ref[...],
                                   (((0,), (0,)), ((), ())),
                                   preferred_element_type=jnp.float32)


def _proj_t_bias_kernel(embt_ref, w_ref, b_ref, out_ref):
    out_ref[...] = lax.dot_general(embt_ref[...], w_ref[...],
                                   (((0,), (0,)), ((), ())),
                                   preferred_element_type=jnp.float32) + b_ref[...]


@functools.lru_cache(maxsize=None)
def _make_sc_gather(batch: int, seq: int):
    assert batch == _NW * _BPW and seq == _L
    tok_per_w = _BPW * _L
    mesh = plsc.VectorSubcoreMesh(core_axis_name="c", subcore_axis_name="s")

    _ST = 65

    @functools.partial(
        pl.kernel,
        out_type=jax.ShapeDtypeStruct((_L, _D // 8, _NW, 8 * _BPW), jnp.float32),
        mesh=mesh,
        compiler_params=pltpu.CompilerParams(use_tc_tiling_on_sc=False,
                                             needs_layout_passes=False),
        scratch_types=[
            pltpu.VMEM((tok_per_w,), jnp.int32),
            pltpu.VMEM((_BPW,), jnp.int32),
            pltpu.VMEM((2, _BPW), jnp.int32),
            pltpu.VMEM((2, _BPW), jnp.int32),
            pltpu.VMEM((2, _BPW, _D), jnp.float32),
            pltpu.VMEM((2, _BPW, _D), jnp.float32),
            pltpu.VMEM((_BPW * _ST,), jnp.float32),
            pltpu.VMEM((2, _D * _BPW), jnp.float32),
            pltpu.SemaphoreType.DMA,
            pltpu.SemaphoreType.DMA,
            pltpu.SemaphoreType.DMA,
            pltpu.SemaphoreType.DMA,
        ],
    )
    def sc_gather(tid_hbm, p1_hbm, p2_hbm, out_hbm,
                  tid_v, bl_v, s1b, s2b, g1b, g2b, sump, stg,
                  sg0, sg1, sw0, sw1):
        wid = lax.axis_index("s") * 2 + lax.axis_index("c")
        sems_g = (sg0, sg1)
        sems_w = (sw0, sw1)

        pltpu.sync_copy(tid_hbm.at[pl.ds(wid * tok_per_w, tok_per_w)], tid_v)
        for k in range(_BPW // 16):
            bl_v[pl.ds(k * 16, 16)] = (lax.iota(jnp.int32, 16) + k * 16) * _L

        def extract(l, par):
            for k in range(_BPW // 16):
                sl = pl.ds(k * 16, 16)
                t = plsc.load_gather(tid_v, [bl_v[sl] + l])
                s1b[par, sl] = lax.shift_right_logical(t, _S2_BITS)
                s2b[par, sl] = (t & _S2_MASK) * 2

        def start_gather(par):
            pltpu.async_copy(p1_hbm.at[s1b.at[par]], g1b.at[par], sems_g[par])
            pltpu.async_copy(p2_hbm.at[s2b.at[par]], g2b.at[par], sems_g[par])

        def wait_gather(par):
            pltpu.make_async_copy(p1_hbm.at[s1b.at[par]], g1b.at[par],
                                  sems_g[par]).wait()
            pltpu.make_async_copy(p2_hbm.at[s2b.at[par]], g2b.at[par],
                                  sems_g[par]).wait()

        def repack(par):
            @plsc.parallel_loop(0, _BPW, unroll=4)
            def _(r):
                for k in range(_D // 16):
                    sl = pl.ds(k * 16, 16)
                    sump[pl.ds(r * _ST + k * 16, 16)] = (
                        g1b[par, r, sl] + g2b[par, r, sl])

        def issue_writes(l, par):
            for jg in range(_D // 8):
                pltpu.async_copy(
                    stg.at[par].at[pl.ds(jg * 8 * _BPW, 8 * _BPW)],
                    out_hbm.at[l, jg, wid], sems_w[par])

        def wait_writes(l, par):
            for jg in range(_D // 8):
                pltpu.make_async_copy(
                    stg.at[par].at[pl.ds(jg * 8 * _BPW, 8 * _BPW)],
                    out_hbm.at[l, jg, wid], sems_w[par]).wait()

        def assemble(l, par):
            iota = lax.iota(jnp.int32, 16)

            @plsc.parallel_loop(0, _D, unroll=4)
            def _(j):
                jsp = jnp.full((16,), j, jnp.int32)
                for bs in range(_BPW // 16):
                    idxv = (iota + bs * 16) * _ST + jsp
                    stg[par, pl.ds(j * _BPW + bs * 16, 16)] = (
                        plsc.load_gather(sump, [idxv]))

        extract(0, 0)
        start_gather(0)

        def pair_body(lp, carry):
            l0 = lp * 2
            l1 = l0 + 1
            wait_gather(0)
            extract(l1, 1)
            start_gather(1)
            repack(0)

            @pl.when(lp > 0)
            def _():
                wait_writes(l0, 0)

            assemble(l0, 0)
            issue_writes(l0, 0)

            wait_gather(1)

            @pl.when(lp < _L // 2 - 1)
            def _():
                extract(l1 + 1, 0)
                start_gather(0)

            repack(1)

            @pl.when(lp > 0)
            def _():
                wait_writes(l1, 1)

            assemble(l1, 1)
            issue_writes(l1, 1)
            return carry

        lax.fori_loop(0, _L // 2, pair_body, 0)
        wait_writes(_L - 2, 0)
        wait_writes(_L - 1, 1)

    return sc_gather


def kernel(token_ids, emb_s1, emb_s2, W, b):
    B, L = token_ids.shape
    n = B * L
    n1 = emb_s1.shape[0]
    n2 = emb_s2.shape[0]
    scale = math.sqrt(_D)
    w1 = W[:_D] * scale
    w2 = jnp.concatenate([W[_D:] * scale,
                          jnp.zeros((_D, _D), jnp.float32)], axis=1)

    p1 = pl.pallas_call(
        _proj_t_bias_kernel,
        out_shape=jax.ShapeDtypeStruct((n1, _D), jnp.float32),
    )(emb_s1.T, w1, b.reshape(1, _D))

    blk = 16384
    p2 = pl.pallas_call(
        _proj_t_kernel,
        grid=(n2 // blk,),
        in_specs=[
            pl.BlockSpec((_D, blk), lambda i: (0, i)),
            pl.BlockSpec((_D, 2 * _D), lambda i: (0, 0)),
        ],
        out_specs=pl.BlockSpec((blk, 2 * _D), lambda i: (i, 0)),
        out_shape=jax.ShapeDtypeStruct((n2, 2 * _D), jnp.float32),
    )(emb_s2.T, w2)

    o = _make_sc_gather(B, L)(
        token_ids.reshape(n),
        p1,
        p2.reshape(n2 * 2, _D),
    )
    o5 = o.reshape(_L, _D // 8, _NW, 8, _BPW)
    return o5.transpose(2, 4, 0, 1, 3).reshape(B, L, _D)


# speedup vs baseline: 1.0431x; 1.0044x over previous
import functools
import math

import jax
import jax.numpy as jnp
from jax import lax
from jax.experimental import pallas as pl
from jax.experimental.pallas import tpu as pltpu
from jax.experimental.pallas import tpu_sc as plsc

_D = 64
_S2_BITS = 20
_S2_MASK = (1 << _S2_BITS) - 1
_NW = 32
_L = 200
_BPW = 128


def _proj_t_kernel(embt_ref, w_ref, out_ref):
    out_ref[...] = lax.dot_general(embt_ref[...], w_ref[...],
                                   (((0,), (0,)), ((), ())),
                                   preferred_element_type=jnp.float32)


def _proj_t_bias_kernel(embt_ref, w_ref, b_ref, out_ref):
    out_ref[...] = lax.dot_general(embt_ref[...], w_ref[...],
                                   (((0,), (0,)), ((), ())),
                                   preferred_element_type=jnp.float32) + b_ref[...]


@functools.lru_cache(maxsize=None)
def _make_sc_gather(batch: int, seq: int):
    assert batch == _NW * _BPW and seq == _L
    tok_per_w = _BPW * _L
    mesh = plsc.VectorSubcoreMesh(core_axis_name="c", subcore_axis_name="s")

    _ST = 65

    @functools.partial(
        pl.kernel,
        out_type=jax.ShapeDtypeStruct((_L, _D // 8, _NW, 8 * _BPW), jnp.float32),
        mesh=mesh,
        compiler_params=pltpu.CompilerParams(use_tc_tiling_on_sc=False,
                                             needs_layout_passes=False),
        scratch_types=[
            pltpu.VMEM((tok_per_w,), jnp.int32),
            pltpu.VMEM((_BPW,), jnp.int32),
            pltpu.VMEM((2, _BPW), jnp.int32),
            pltpu.VMEM((2, _BPW), jnp.int32),
            pltpu.VMEM((2, _BPW, _D), jnp.float32),
            pltpu.VMEM((2, _BPW, _D), jnp.float32),
            pltpu.VMEM((_BPW * _ST,), jnp.float32),
            pltpu.VMEM((2, _D * _BPW), jnp.float32),
            pltpu.SemaphoreType.DMA,
            pltpu.SemaphoreType.DMA,
            pltpu.SemaphoreType.DMA,
            pltpu.SemaphoreType.DMA,
        ],
    )
    def sc_gather(tid_hbm, p1_hbm, p2_hbm, out_hbm,
                  tid_v, bl_v, s1b, s2b, g1b, g2b, sump, stg,
                  sg0, sg1, sw0, sw1):
        wid = lax.axis_index("s") * 2 + lax.axis_index("c")
        sems_g = (sg0, sg1)
        sems_w = (sw0, sw1)

        pltpu.sync_copy(tid_hbm.at[pl.ds(wid * tok_per_w, tok_per_w)], tid_v)
        for k in range(_BPW // 16):
            bl_v[pl.ds(k * 16, 16)] = (lax.iota(jnp.int32, 16) + k * 16) * _L

        def extract(l, par):
            for k in range(_BPW // 16):
                sl = pl.ds(k * 16, 16)
                t = plsc.load_gather(tid_v, [bl_v[sl] + l])
                s1b[par, sl] = lax.shift_right_logical(t, _S2_BITS)
                s2b[par, sl] = (t & _S2_MASK) * 2

        def start_gather(par):
            pltpu.async_copy(p1_hbm.at[s1b.at[par]], g1b.at[par], sems_g[par])
            pltpu.async_copy(p2_hbm.at[s2b.at[par]], g2b.at[par], sems_g[par])

        def wait_gather(par):
            pltpu.make_async_copy(p1_hbm.at[s1b.at[par]], g1b.at[par],
                                  sems_g[par]).wait()
            pltpu.make_async_copy(p2_hbm.at[s2b.at[par]], g2b.at[par],
                                  sems_g[par]).wait()

        def repack(par):
            @plsc.parallel_loop(0, _BPW, unroll=4)
            def _(r):
                for k in range(_D // 16):
                    sl = pl.ds(k * 16, 16)
                    sump[pl.ds(r * _ST + k * 16, 16)] = (
                        g1b[par, r, sl] + g2b[par, r, sl])

        def issue_writes(l, par):
            for jg in range(_D // 8):
                pltpu.async_copy(
                    stg.at[par].at[pl.ds(jg * 8 * _BPW, 8 * _BPW)],
                    out_hbm.at[l, jg, wid], sems_w[par])

        def wait_writes(l, par):
            for jg in range(_D // 8):
                pltpu.make_async_copy(
                    stg.at[par].at[pl.ds(jg * 8 * _BPW, 8 * _BPW)],
                    out_hbm.at[l, jg, wid], sems_w[par]).wait()

        def assemble(l, par):
            iota = lax.iota(jnp.int32, 16)

            @plsc.parallel_loop(0, _D, unroll=4)
            def _(j):
                jsp = jnp.full((16,), j, jnp.int32)
                for bs in range(_BPW // 16):
                    idxv = (iota + bs * 16) * _ST + jsp
                    stg[par, pl.ds(j * _BPW + bs * 16, 16)] = (
                        plsc.load_gather(sump, [idxv]))

        extract(0, 0)
        start_gather(0)

        def pair_body(lp, carry):
            l0 = lp * 2
            l1 = l0 + 1
            wait_gather(0)
            extract(l1, 1)
            start_gather(1)
            repack(0)

            @pl.when(lp > 0)
            def _():
                wait_writes(l0, 0)

            assemble(l0, 0)
            issue_writes(l0, 0)

            wait_gather(1)

            @pl.when(lp < _L // 2 - 1)
            def _():
                extract(l1 + 1, 0)
                start_gather(0)

            repack(1)

            @pl.when(lp > 0)
            def _():
                wait_writes(l1, 1)

            assemble(l1, 1)
            issue_writes(l1, 1)
            return carry

        lax.fori_loop(0, _L // 2, pair_body, 0)
        wait_writes(_L - 2, 0)
        wait_writes(_L - 1, 1)

    return sc_gather


def kernel(token_ids, emb_s1, emb_s2, W, b):
    B, L = token_ids.shape
    n = B * L
    n1 = emb_s1.shape[0]
    n2 = emb_s2.shape[0]
    scale = math.sqrt(_D)
    w1 = W[:_D] * scale
    w2 = jnp.concatenate([W[_D:] * scale,
                          jnp.zeros((_D, _D), jnp.float32)], axis=1)

    p1 = pl.pallas_call(
        _proj_t_bias_kernel,
        out_shape=jax.ShapeDtypeStruct((n1, _D), jnp.float32),
    )(emb_s1.T, w1, b.reshape(1, _D))

    blk = 32768
    p2 = pl.pallas_call(
        _proj_t_kernel,
        grid=(n2 // blk,),
        in_specs=[
            pl.BlockSpec((_D, blk), lambda i: (0, i)),
            pl.BlockSpec((_D, 2 * _D), lambda i: (0, 0)),
        ],
        out_specs=pl.BlockSpec((blk, 2 * _D), lambda i: (i, 0)),
        out_shape=jax.ShapeDtypeStruct((n2, 2 * _D), jnp.float32),
    )(emb_s2.T, w2)

    o = _make_sc_gather(B, L)(
        token_ids.reshape(n),
        p1,
        p2.reshape(n2 * 2, _D),
    )
    o5 = o.reshape(_L, _D // 8, _NW, 8, _BPW)
    return o5.transpose(2, 4, 0, 1, 3).reshape(B, L, _D)
